# baseline (speedup 1.0000x reference)
import functools

import jax
import jax.numpy as jnp
from jax import lax
from jax.experimental import pallas as pl
from jax.experimental.pallas import tpu as pltpu
from jax.experimental.pallas import tpu_sc as plsc

EMBED = 64
VOCAB = 1000000
NC = 2
NS = 16
NW = NC * NS
CHUNK = 128
LANES = 16
GROUPS = CHUNK // LANES
VBLK = 128
NPAIR = VOCAB // 2
NFULL = VOCAB // VBLK
TAILV = NFULL * VBLK
SKEW = VBLK + 1


def _phase_a_kernel():
    mesh = plsc.VectorSubcoreMesh(core_axis_name="c", subcore_axis_name="s")
    n_slots = 246

    @functools.partial(
        pl.kernel,
        mesh=mesh,
        out_type=jax.ShapeDtypeStruct((NPAIR, 2 * EMBED), jnp.float32),
        scratch_types=[
            pltpu.VMEM((EMBED, SKEW), jnp.float32),
            pltpu.VMEM((EMBED, SKEW), jnp.float32),
            pltpu.VMEM((EMBED, 2 * EMBED), jnp.float32),
            pltpu.VMEM((EMBED, 2 * EMBED), jnp.float32),
            pltpu.SemaphoreType.DMA,
            pltpu.SemaphoreType.DMA,
        ],
        compiler_params=pltpu.CompilerParams(needs_layout_passes=False),
    )
    def reformat(wt_hbm, wtail_hbm, out_hbm, st_a, st_b, ob_a, ob_b,
                 isem, osem):
        wid = lax.axis_index("s") * NC + lax.axis_index("c")
        iota = lax.iota(jnp.int32, LANES)

        def blk(s):
            return jnp.minimum(wid + NW * s, NFULL - 1)

        def start_in(c, st):
            off = pl.multiple_of(c * VBLK, VBLK)
            pltpu.async_copy(wt_hbm.at[:, pl.ds(off, VBLK)],
                             st.at[:, pl.ds(0, VBLK)], isem)

        def wait_in(st):
            pltpu.make_async_copy(wt_hbm.at[:, pl.ds(0, VBLK)],
                                  st.at[:, pl.ds(0, VBLK)], isem).wait()

        def transpose(st, ob, nq, voff):
            for q in range(nq):
                for h in range(2):
                    ci = jnp.full((LANES,), voff + 2 * q + h, jnp.int32)
                    for dq in range(EMBED // LANES):
                        ri = iota + dq * LANES
                        v = plsc.load_gather(st, [ri, ci])
                        ob[q, pl.ds(h * EMBED + dq * LANES, LANES)] = v

        def start_out(c, ob):
            off = pl.multiple_of(c * (VBLK // 2), VBLK // 2)
            pltpu.async_copy(ob, out_hbm.at[pl.ds(off, VBLK // 2)], osem)

        def wait_out(ob):
            pltpu.make_async_copy(ob, out_hbm.at[pl.ds(0, VBLK // 2)],
                                  osem).wait()

        start_in(blk(0), st_a)
        start_in(blk(1), st_b)

        def body(k, carry):
            s0 = 2 * k
            c0 = blk(s0)
            c1 = blk(s0 + 1)

            wait_in(st_a)

            @pl.when(k > 0)
            def _():
                wait_out(ob_a)
                wait_out(ob_b)

            transpose(st_a, ob_a, VBLK // 2, 0)
            start_out(c0, ob_a)
            start_in(blk(s0 + 2), st_a)

            wait_in(st_b)
            transpose(st_b, ob_b, VBLK // 2, 0)
            start_out(c1, ob_b)
            start_in(blk(s0 + 3), st_b)
            return carry

        lax.fori_loop(0, n_slots // 2, body, 0)
        wait_in(st_a)
        wait_in(st_b)
        wait_out(ob_a)
        wait_out(ob_b)

        @pl.when(wid == 0)
        def _():
            pltpu.sync_copy(wtail_hbm, ob_a.at[pl.ds(0, VBLK // 4)])
            pltpu.sync_copy(ob_a.at[pl.ds(0, VBLK // 4)],
                            out_hbm.at[pl.ds(TAILV // 2, VBLK // 4)])

    return reformat


def _phase_b_kernel(n_rows):
    n_per_w = n_rows // NW
    n_chunks = n_per_w // CHUNK

    mesh = plsc.VectorSubcoreMesh(core_axis_name="c", subcore_axis_name="s")

    @functools.partial(
        pl.kernel,
        mesh=mesh,
        out_type=jax.ShapeDtypeStruct((n_rows, EMBED), jnp.float32),
        scratch_types=[
            pltpu.VMEM((n_chunks, CHUNK), jnp.int32),
            pltpu.VMEM((n_chunks, CHUNK), jnp.int32),
            pltpu.VMEM((CHUNK, 2 * EMBED), jnp.float32),
            pltpu.VMEM((CHUNK, 2 * EMBED), jnp.float32),
            pltpu.VMEM((CHUNK, EMBED), jnp.float32),
            pltpu.VMEM((CHUNK, EMBED), jnp.float32),
            pltpu.SemaphoreType.DMA,
            pltpu.SemaphoreType.DMA,
        ],
        compiler_params=pltpu.CompilerParams(needs_layout_passes=False),
    )
    def lookup(p_hbm, h_hbm, wp_hbm, out_hbm,
               pv, hv, rows_a, rows_b, out_a, out_b, gsem, osem):
        wid = lax.axis_index("s") * NC + lax.axis_index("c")
        cbase = wid * n_chunks
        base = wid * n_per_w

        pltpu.sync_copy(p_hbm.at[pl.ds(cbase, n_chunks)], pv)
        pltpu.sync_copy(h_hbm.at[pl.ds(cbase, n_chunks)], hv)

        def start_gather(c, rbuf):
            pltpu.async_copy(wp_hbm.at[pv.at[c]], rbuf, gsem)

        def wait_gather(rbuf):
            pltpu.make_async_copy(wp_hbm.at[pv.at[0]], rbuf, gsem).wait()

        def wait_out(obuf):
            pltpu.make_async_copy(obuf, out_hbm.at[pl.ds(base, CHUNK)],
                                  osem).wait()

        def compact(c, rbuf, obuf):
            for g in range(GROUPS):
                hvec = hv[c, pl.ds(g * LANES, LANES)]
                for k in range(LANES):
                    i = g * LANES + k
                    h64 = hvec[k]
                    for q in range(EMBED // LANES):
                        obuf[i, pl.ds(q * LANES, LANES)] = (
                            rbuf[i, pl.ds(h64 + q * LANES, LANES)]
                        )

        def emit_out(c, obuf):
            pltpu.async_copy(obuf, out_hbm.at[pl.ds(base + c * CHUNK, CHUNK)],
                             osem)

        start_gather(0, rows_a)

        def body(j, carry):
            c0 = 2 * j
            c1 = c0 + 1

            wait_gather(rows_a)
            start_gather(c1, rows_b)

            @pl.when(j > 0)
            def _():
                wait_out(out_a)
                wait_out(out_b)

            compact(c0, rows_a, out_a)
            emit_out(c0, out_a)

            wait_gather(rows_b)

            @pl.when(c1 + 1 < n_chunks)
            def _():
                start_gather(c1 + 1, rows_a)

            compact(c1, rows_b, out_b)
            emit_out(c1, out_b)
            return carry

        lax.fori_loop(0, n_chunks // 2, body, 0)
        wait_out(out_a)
        wait_out(out_b)

    return lookup


@functools.lru_cache(maxsize=None)
def _kernels(n_rows):
    return _phase_a_kernel(), _phase_b_kernel(n_rows)


def kernel(x, weight):
    b, s = x.shape
    n_rows = b * s
    xf = x.reshape(n_rows).astype(jnp.int32)
    p2d = lax.shift_right_logical(xf, 1).reshape(n_rows // CHUNK, CHUNK)
    h2d = lax.shift_left(jnp.bitwise_and(xf, 1), 6).reshape(
        n_rows // CHUNK, CHUNK)
    reformat, lookup = _kernels(n_rows)
    wtail = weight[TAILV:, :].reshape(VBLK // 4, 2 * EMBED)
    wp = reformat(weight.T, wtail)
    out = lookup(p2d, h2d, wp)
    return out.reshape(b, s, EMBED)


# speedup vs baseline: 1.8088x; 1.8088x over previous
import functools

import jax
import jax.numpy as jnp
from jax import lax
from jax.experimental import pallas as pl
from jax.experimental.pallas import tpu as pltpu
from jax.experimental.pallas import tpu_sc as plsc

EMBED = 64
VOCAB = 1000000
NC = 2
NS = 16
NW = NC * NS
CHUNK = 128
LANES = 16
GROUPS = CHUNK // LANES
NPAIR = VOCAB // 2
TBLK = 1024
OBLK = TBLK // 2
TGRID = (VOCAB + TBLK - 1) // TBLK


def _phase_a_call(wt):
    def tk(wt_ref, out_ref):
        t = wt_ref[...].T
        out_ref[...] = jnp.concatenate([t, t], axis=1)

    return pl.pallas_call(
        tk,
        grid=(TGRID,),
        in_specs=[pl.BlockSpec((EMBED, TBLK), lambda j: (0, j))],
        out_specs=pl.BlockSpec((TBLK, 2 * EMBED), lambda j: (j, 0)),
        out_shape=jax.ShapeDtypeStruct((VOCAB, 2 * EMBED), jnp.float32),
    )(wt)


def _phase_b_kernel(n_rows):
    n_per_w = n_rows // NW
    n_chunks = n_per_w // CHUNK

    mesh = plsc.VectorSubcoreMesh(core_axis_name="c", subcore_axis_name="s")

    @functools.partial(
        pl.kernel,
        mesh=mesh,
        out_type=jax.ShapeDtypeStruct((n_rows, EMBED), jnp.float32),
        scratch_types=[
            pltpu.VMEM((n_chunks, CHUNK), jnp.int32),
            pltpu.VMEM((n_chunks, CHUNK), jnp.int32),
            pltpu.VMEM((CHUNK, 2 * EMBED), jnp.float32),
            pltpu.VMEM((CHUNK, 2 * EMBED), jnp.float32),
            pltpu.VMEM((CHUNK, EMBED), jnp.float32),
            pltpu.VMEM((CHUNK, EMBED), jnp.float32),
            pltpu.SemaphoreType.DMA,
            pltpu.SemaphoreType.DMA,
        ],
        compiler_params=pltpu.CompilerParams(needs_layout_passes=False),
    )
    def lookup(p_hbm, h_hbm, wp_hbm, out_hbm,
               pv, hv, rows_a, rows_b, out_a, out_b, gsem, osem):
        wid = lax.axis_index("s") * NC + lax.axis_index("c")
        cbase = wid * n_chunks
        base = wid * n_per_w

        pltpu.sync_copy(p_hbm.at[pl.ds(cbase, n_chunks)], pv)
        pltpu.sync_copy(h_hbm.at[pl.ds(cbase, n_chunks)], hv)

        def start_gather(c, rbuf):
            pltpu.async_copy(wp_hbm.at[pv.at[c]], rbuf, gsem)

        def wait_gather(rbuf):
            pltpu.make_async_copy(wp_hbm.at[pv.at[0]], rbuf, gsem).wait()

        def wait_out(obuf):
            pltpu.make_async_copy(obuf, out_hbm.at[pl.ds(base, CHUNK)],
                                  osem).wait()

        def compact(c, rbuf, obuf):
            for g in range(GROUPS):
                hvec = hv[c, pl.ds(g * LANES, LANES)]
                for k in range(LANES):
                    i = g * LANES + k
                    h64 = hvec[k]
                    for q in range(EMBED // LANES):
                        obuf[i, pl.ds(q * LANES, LANES)] = (
                            rbuf[i, pl.ds(h64 + q * LANES, LANES)]
                        )

        def emit_out(c, obuf):
            pltpu.async_copy(obuf, out_hbm.at[pl.ds(base + c * CHUNK, CHUNK)],
                             osem)

        start_gather(0, rows_a)

        def body(j, carry):
            c0 = 2 * j
            c1 = c0 + 1

            wait_gather(rows_a)
            start_gather(c1, rows_b)

            @pl.when(j > 0)
            def _():
                wait_out(out_a)
                wait_out(out_b)

            compact(c0, rows_a, out_a)
            emit_out(c0, out_a)

            wait_gather(rows_b)

            @pl.when(c1 + 1 < n_chunks)
            def _():
                start_gather(c1 + 1, rows_a)

            compact(c1, rows_b, out_b)
            emit_out(c1, out_b)
            return carry

        lax.fori_loop(0, n_chunks // 2, body, 0)
        wait_out(out_a)
        wait_out(out_b)

    return lookup


@functools.lru_cache(maxsize=None)
def _kernels(n_rows):
    return _phase_b_kernel(n_rows)


def kernel(x, weight):
    b, s = x.shape
    n_rows = b * s
    xf = x.reshape(n_rows).astype(jnp.int32)
    p2d = xf.reshape(n_rows // CHUNK, CHUNK)
    h2d = jnp.zeros_like(p2d)
    lookup = _kernels(n_rows)
    wp = _phase_a_call(weight.T)
    out = lookup(p2d, h2d, wp)
    return out.reshape(b, s, EMBED)


# speedup vs baseline: 2.4516x; 1.3554x over previous
import functools

import jax
import jax.numpy as jnp
from jax import lax
from jax.experimental import pallas as pl
from jax.experimental.pallas import tpu as pltpu
from jax.experimental.pallas import tpu_sc as plsc

EMBED = 64
NC = 2
NS = 16
NW = NC * NS
CHUNK = 128


@functools.lru_cache(maxsize=None)
def _make_kernel(n_rows: int):
    n_per_w = n_rows // NW
    n_chunks = n_per_w // CHUNK

    mesh = plsc.VectorSubcoreMesh(core_axis_name="c", subcore_axis_name="s")

    @functools.partial(
        pl.kernel,
        mesh=mesh,
        out_type=jax.ShapeDtypeStruct((n_rows, EMBED), jnp.float32),
        scratch_types=[
            pltpu.VMEM((n_chunks, CHUNK), jnp.int32),
            pltpu.VMEM((CHUNK, EMBED), jnp.float32),
            pltpu.VMEM((CHUNK, EMBED), jnp.float32),
            pltpu.SemaphoreType.DMA,
            pltpu.SemaphoreType.DMA,
        ],
        compiler_params=pltpu.CompilerParams(use_tc_tiling_on_sc=False),
    )
    def emb_kernel(idx_hbm, table_hbm, out_hbm, idx_v, rows_a, rows_b,
                   gsem, osem):
        wid = lax.axis_index("s") * NC + lax.axis_index("c")
        pltpu.sync_copy(idx_hbm.at[pl.ds(wid * n_chunks, n_chunks)], idx_v)
        base = wid * n_per_w

        def start_gather(c, rbuf):
            pltpu.async_copy(table_hbm.at[idx_v.at[c]], rbuf, gsem)

        def wait_gather(rbuf):
            pltpu.make_async_copy(table_hbm.at[idx_v.at[0]], rbuf,
                                  gsem).wait()

        def emit_out(c, rbuf):
            pltpu.async_copy(rbuf, out_hbm.at[pl.ds(base + c * CHUNK, CHUNK)],
                             osem)

        def wait_out(rbuf):
            pltpu.make_async_copy(rbuf, out_hbm.at[pl.ds(base, CHUNK)],
                                  osem).wait()

        start_gather(0, rows_a)

        def body(j, carry):
            c0 = 2 * j
            c1 = c0 + 1

            wait_gather(rows_a)

            @pl.when(j > 0)
            def _():
                wait_out(rows_b)

            start_gather(c1, rows_b)
            emit_out(c0, rows_a)

            wait_gather(rows_b)
            wait_out(rows_a)

            @pl.when(c1 + 1 < n_chunks)
            def _():
                start_gather(c1 + 1, rows_a)

            emit_out(c1, rows_b)
            return carry

        lax.fori_loop(0, n_chunks // 2, body, 0)
        wait_out(rows_b)

    return emb_kernel


def kernel(x, weight):
    b, s = x.shape
    n_rows = b * s
    idx = x.reshape(n_rows // CHUNK, CHUNK).astype(jnp.int32)
    out = _make_kernel(n_rows)(idx, weight)
    return out.reshape(b, s, EMBED)
